# Initial kernel scaffold; baseline (speedup 1.0000x reference)
import jax
import jax.numpy as jnp
from jax.experimental import pallas as pl


def kernel(sites, bonds, states, graph_to_sites, graph_to_bonds, W1, b1, W2, b2, W3, b3):
    raise NotImplementedError("write your pallas kernel here")


# speedup vs baseline: 5.0748x; 5.0748x over previous
import functools

import jax
import jax.numpy as jnp
from jax.experimental import pallas as pl


def _segsum_body(ids_ref, data_ref, sums_ref, cnts_ref, *, W, C, D):
    @pl.when(pl.program_id(0) == 0)
    def _():
        sums_ref[...] = jnp.zeros_like(sums_ref)
        cnts_ref[...] = jnp.zeros_like(cnts_ref)

    ids = ids_ref[0]
    chunk = data_ref[...]
    lo = jnp.min(ids)
    hi = jnp.max(ids)
    w_init = (lo // 8) * 8

    def cond(w0):
        return w0 <= hi

    def step(w0):
        w0a = pl.multiple_of(w0, 8)
        wrows = w0a + jax.lax.broadcasted_iota(jnp.int32, (W, C), 0)
        oh = (jnp.broadcast_to(ids, (W, C)) == wrows).astype(jnp.float32)
        psum = jax.lax.dot_general(
            oh, chunk, (((1,), (0,)), ((), ())),
            preferred_element_type=jnp.float32)
        pcnt = jnp.sum(oh, axis=1, keepdims=True)
        sums_ref[pl.ds(w0a, W), :] += psum
        cnts_ref[pl.ds(w0a, W), :] += jnp.broadcast_to(pcnt, (W, 8))
        return w0a + W

    jax.lax.while_loop(cond, step, w_init)


def _segment_sums(data, ids, num_segments, block_rows, window):
    n, d = data.shape
    assert n % block_rows == 0, (n, block_rows)
    grid = n // block_rows
    g_pad = num_segments + window
    ids3 = ids.reshape(grid, 1, block_rows)
    sums, cnts = pl.pallas_call(
        functools.partial(_segsum_body, W=window, C=block_rows, D=d),
        grid=(grid,),
        in_specs=[
            pl.BlockSpec((1, 1, block_rows), lambda c: (c, 0, 0)),
            pl.BlockSpec((block_rows, d), lambda c: (c, 0)),
        ],
        out_specs=[
            pl.BlockSpec((g_pad, d), lambda c: (0, 0)),
            pl.BlockSpec((g_pad, 8), lambda c: (0, 0)),
        ],
        out_shape=[
            jax.ShapeDtypeStruct((g_pad, d), jnp.float32),
            jax.ShapeDtypeStruct((g_pad, 8), jnp.float32),
        ],
    )(ids3, data)
    return sums[:num_segments], cnts[:num_segments, 0:1]


def _mlp_body(pb_ref, cb_ref, ps_ref, cs_ref, st_ref,
              w1b_ref, w1s_ref, w1t_ref, b1_ref,
              w2_ref, b2_ref, w3_ref, b3_ref, out_ref):
    bp = pb_ref[...] / jnp.maximum(cb_ref[...], 1.0)
    sp = ps_ref[...] / jnp.maximum(cs_ref[...], 1.0)

    def dot(a, b):
        return jax.lax.dot_general(a, b, (((1,), (0,)), ((), ())),
                                   preferred_element_type=jnp.float32)

    h = dot(bp, w1b_ref[...]) + dot(sp, w1s_ref[...]) + dot(st_ref[...], w1t_ref[...])
    h = jnp.maximum(h + b1_ref[...], 0.0)
    h = jnp.maximum(dot(h, w2_ref[...]) + b2_ref[...], 0.0)
    out_ref[...] = jnp.maximum(dot(h, w3_ref[...]) + b3_ref[...], 0.0)


def kernel(sites, bonds, states, graph_to_sites, graph_to_bonds,
           W1, b1, W2, b2, W3, b3):
    num_graphs, state_len = states.shape
    bond_len = bonds.shape[1]
    site_len = sites.shape[1]

    def pick_block(n, cap):
        for c in range(cap, 7, -8):
            if n % c == 0:
                return c
        return n

    bsum, bcnt = _segment_sums(bonds, graph_to_bonds, num_graphs,
                               pick_block(bonds.shape[0], 2048), 8)
    ssum, scnt = _segment_sums(sites, graph_to_sites, num_graphs,
                               pick_block(sites.shape[0], 512), 32)

    w1 = W1.T
    w1b = w1[:bond_len]
    w1s = w1[bond_len:bond_len + site_len]
    w1t = w1[bond_len + site_len:]

    out = pl.pallas_call(
        _mlp_body,
        out_shape=jax.ShapeDtypeStruct((num_graphs, state_len), jnp.float32),
    )(bsum, bcnt, ssum, scnt, states,
      w1b, w1s, w1t, b1.reshape(1, -1),
      W2.T, b2.reshape(1, -1), W3.T, b3.reshape(1, -1))
    return out


# speedup vs baseline: 7.3898x; 1.4562x over previous
import functools

import jax
import jax.numpy as jnp
from jax import lax
from jax.experimental import pallas as pl
from jax.experimental.pallas import tpu as pltpu
from jax.experimental.pallas import tpu_sc as plsc

_G = 4096
_GP = 4224
_ZROWS = _GP // 16
_SCAP = 16


def _zeros16():
    return jnp.zeros((16,), jnp.float32)


def _sink16():
    return jnp.full((16,), _G, jnp.int32)


def _sc_pool_body(data_hbm, ids_hbm, out_hbm, buf, idbuf, zbuf, stage_v,
                  stage_iv, acc_buf, table, *, B, D, W, NB, KMAX):
    cid = lax.axis_index("c")
    sid = lax.axis_index("s")
    tid = cid * 16 + sid
    nvec = D // 16

    def zb(i, _):
        for j in range(W // 16):
            zbuf[i, pl.ds(16 * j, 16)] = _zeros16()
        return 0
    lax.fori_loop(0, _ZROWS, zb, 0)
    pltpu.sync_copy(zbuf, table.at[pl.ds(sid * _ZROWS, _ZROWS)])
    plsc.subcore_barrier()

    iota16 = lax.iota(jnp.int32, 16)
    stage_iv[0, pl.ds(0, 16)] = _sink16()

    def flush(seg, cnt, sn):
        def do(sn):
            for j in range(nvec):
                stage_v[sn, pl.ds(16 * j, 16)] = acc_buf[0, pl.ds(16 * j, 16)]
            stage_v[sn, pl.ds(D, 16)] = jnp.where(
                iota16 == 0, cnt.astype(jnp.float32), 0.0)
            iv = stage_iv[0, pl.ds(0, 16)]
            stage_iv[0, pl.ds(0, 16)] = jnp.where(iota16 == sn, seg, iv)
            sn2 = sn + 1

            def dodma(_):
                pltpu.sync_copy(stage_v, table.at[stage_iv.at[0]], add=True)
                stage_iv[0, pl.ds(0, 16)] = _sink16()
                return jnp.int32(0)
            return lax.cond(sn2 == _SCAP, dodma, lambda _: sn2, 0)
        return lax.cond(cnt > 0, do, lambda s: s, sn)

    def process_block(base, sn):
        pltpu.sync_copy(data_hbm.at[pl.ds(base, B)], buf)
        pltpu.sync_copy(ids_hbm.at[pl.ds(base, B)], idbuf)

        def group_body(g, carry):
            cur, cnt, sn = carry
            i0 = g * 16
            idvec = idbuf[pl.ds(i0, 16)]
            a = idvec[0]
            z = idvec[15]

            def fast(cur, cnt, sn):
                sums = []
                for j in range(nvec):
                    rows = [buf[i0 + r, pl.ds(16 * j, 16)]
                            for r in range(16)]
                    while len(rows) > 1:
                        rows = ([rows[k] + rows[k + 1]
                                 for k in range(0, len(rows) - 1, 2)]
                                + (rows[-1:] if len(rows) % 2 else []))
                    sums.append(rows[0])

                def new_seg(sn):
                    sn2 = flush(cur, cnt, sn)
                    for j in range(nvec):
                        acc_buf[0, pl.ds(16 * j, 16)] = sums[j]
                    return a, jnp.int32(16), sn2

                def same_seg(sn):
                    for j in range(nvec):
                        acc_buf[0, pl.ds(16 * j, 16)] = (
                            acc_buf[0, pl.ds(16 * j, 16)] + sums[j])
                    return cur, cnt + 16, sn
                return lax.cond(a != cur, new_seg, same_seg, sn)

            def slow(cur, cnt, sn):
                for r in range(16):
                    idv = idvec[r]
                    row = tuple(buf[i0 + r, pl.ds(16 * j, 16)]
                                for j in range(nvec))
                    new = idv != cur

                    def donew(sn, cnt=cnt, cur=cur, row=row):
                        sn2 = flush(cur, cnt, sn)
                        for j in range(nvec):
                            acc_buf[0, pl.ds(16 * j, 16)] = row[j]
                        return sn2

                    def doold(sn, row=row):
                        for j in range(nvec):
                            acc_buf[0, pl.ds(16 * j, 16)] = (
                                acc_buf[0, pl.ds(16 * j, 16)] + row[j])
                        return sn
                    sn = lax.cond(new, donew, doold, sn)
                    cnt = jnp.where(new, 1, cnt + 1)
                    cur = idv
                return cur, cnt, sn

            return lax.cond(a == z, fast, slow, cur, cnt, sn)

        cur, cnt, sn = lax.fori_loop(
            0, B // 16, group_body, (jnp.int32(-1), jnp.int32(0), sn))
        return flush(cur, cnt, sn)

    def block_loop(k, sn):
        b = tid + 32 * k
        if NB % 32 == 0:
            return process_block(b * B, sn)
        return lax.cond(b < NB, lambda s: process_block(b * B, s),
                        lambda s: s, sn)

    lax.fori_loop(0, KMAX, block_loop, jnp.int32(0))

    pltpu.sync_copy(stage_v, table.at[stage_iv.at[0]], add=True)

    plsc.subcore_barrier()
    pltpu.sync_copy(table.at[pl.ds(sid * _ZROWS, _ZROWS)],
                    out_hbm.at[cid, pl.ds(sid * _ZROWS, _ZROWS)])


def _sc_pool(data, ids, block_rows):
    n, d = data.shape
    w = d + 16
    nb = n // block_rows
    assert nb * block_rows == n
    kmax = -(-nb // 32)
    mesh = plsc.VectorSubcoreMesh(core_axis_name="c", subcore_axis_name="s")
    body = functools.partial(
        _sc_pool_body, B=block_rows, D=d, W=w, NB=nb, KMAX=kmax)
    return pl.kernel(
        body,
        out_type=jax.ShapeDtypeStruct((2, _GP, w), jnp.float32),
        mesh=mesh,
        compiler_params=pltpu.CompilerParams(use_tc_tiling_on_sc=False),
        scratch_types=[
            pltpu.VMEM((block_rows, d), jnp.float32),
            pltpu.VMEM((block_rows,), jnp.int32),
            pltpu.VMEM((_ZROWS, w), jnp.float32),
            pltpu.VMEM((_SCAP, w), jnp.float32),
            pltpu.VMEM((1, 16), jnp.int32),
            pltpu.VMEM((1, d), jnp.float32),
            pltpu.VMEM_SHARED((_GP, w), jnp.float32),
        ],
    )(data, ids)


def _mlp_body(pb_ref, ps_ref, st_ref,
              w1b_ref, w1s_ref, w1t_ref, b1_ref,
              w2_ref, b2_ref, w3_ref, b3_ref, out_ref):
    pb = pb_ref[0] + pb_ref[1]
    ps = ps_ref[0] + ps_ref[1]
    bp = pb[:_G, :16] / jnp.maximum(pb[:_G, 16:17], 1.0)
    sp = ps[:_G, :128] / jnp.maximum(ps[:_G, 128:129], 1.0)

    def dot(a, b):
        return jax.lax.dot_general(a, b, (((1,), (0,)), ((), ())),
                                   preferred_element_type=jnp.float32)

    h = dot(bp, w1b_ref[...]) + dot(sp, w1s_ref[...]) + dot(st_ref[...], w1t_ref[...])
    h = jnp.maximum(h + b1_ref[...], 0.0)
    h = jnp.maximum(dot(h, w2_ref[...]) + b2_ref[...], 0.0)
    out_ref[...] = jnp.maximum(dot(h, w3_ref[...]) + b3_ref[...], 0.0)


def kernel(sites, bonds, states, graph_to_sites, graph_to_bonds,
           W1, b1, W2, b2, W3, b3):
    num_graphs, state_len = states.shape
    bond_len = bonds.shape[1]
    site_len = sites.shape[1]
    assert num_graphs == _G

    pb = _sc_pool(bonds, graph_to_bonds, 2000)
    ps = _sc_pool(sites, graph_to_sites, 400)

    w1 = W1.T
    w1b = w1[:bond_len]
    w1s = w1[bond_len:bond_len + site_len]
    w1t = w1[bond_len + site_len:]

    out = pl.pallas_call(
        _mlp_body,
        out_shape=jax.ShapeDtypeStruct((num_graphs, state_len), jnp.float32),
    )(pb, ps, states,
      w1b, w1s, w1t, b1.reshape(1, -1),
      W2.T, b2.reshape(1, -1), W3.T, b3.reshape(1, -1))
    return out


# speedup vs baseline: 7.5015x; 1.0151x over previous
import functools

import jax
import jax.numpy as jnp
from jax import lax
from jax.experimental import pallas as pl
from jax.experimental.pallas import tpu as pltpu
from jax.experimental.pallas import tpu_sc as plsc

_G = 4096
_GP = 4224
_ZROWS = _GP // 16
_ZCHUNK = _ZROWS // 11
_SCAP = 16
_W = 128


def _zeros16():
    return jnp.zeros((16,), jnp.float32)


def _sink16():
    return jnp.full((16,), _G, jnp.int32)


def _sc_pool_body(*refs, BP, BR, D, KMAX, CNT, CBR):
    if CBR:
        (data_hbm, ids_hbm, cids_hbm, out_hbm, bufA, bufB, idbufA, idbufB,
         zbuf, stage_v, stage_iv, acc_buf, table, semA, semB) = refs
    else:
        (data_hbm, ids_hbm, out_hbm, bufA, bufB, idbufA, idbufB,
         zbuf, stage_v, stage_iv, acc_buf, table, semA, semB) = refs
    cid = lax.axis_index("c")
    sid = lax.axis_index("s")
    tid = cid * 16 + sid
    nvec = D // 16

    def zb(i, _):
        for j in range(_W // 16):
            zbuf[i, pl.ds(16 * j, 16)] = _zeros16()
        return 0
    lax.fori_loop(0, _ZCHUNK, zb, 0)
    for i in range(11):
        pltpu.sync_copy(zbuf,
                        table.at[pl.ds(sid * _ZROWS + i * _ZCHUNK, _ZCHUNK)])
    for s in range(_SCAP):
        for j in range(_W // 16):
            stage_v[s, pl.ds(16 * j, 16)] = _zeros16()
    stage_iv[0, pl.ds(0, 16)] = _sink16()
    plsc.subcore_barrier()

    iota16 = lax.iota(jnp.int32, 16)

    def push(seg, sn, write):
        write(sn)
        iv = stage_iv[0, pl.ds(0, 16)]
        stage_iv[0, pl.ds(0, 16)] = jnp.where(iota16 == sn, seg, iv)
        sn2 = sn + 1

        def dodma(_):
            pltpu.sync_copy(stage_v, table.at[stage_iv.at[0]], add=True)
            stage_iv[0, pl.ds(0, 16)] = _sink16()
            return jnp.int32(0)
        return lax.cond(sn2 == _SCAP, dodma, lambda _: sn2, 0)

    def flush(seg, cnt, sn):
        def do(sn):
            def write(sn):
                for j in range(nvec):
                    stage_v[sn, pl.ds(16 * j, 16)] = acc_buf[0, pl.ds(16 * j, 16)]
                if CNT:
                    stage_v[sn, pl.ds(D, 16)] = jnp.where(
                        iota16 == 0, cnt.astype(jnp.float32), 0.0)
            return push(seg, sn, write)
        return lax.cond(cnt > 0, do, lambda s: s, sn)

    def cflush(seg, cnt, sn):
        def do(sn):
            def write(sn):
                stage_v[sn, pl.ds(D, 16)] = jnp.where(
                    iota16 == 1, cnt.astype(jnp.float32), 0.0)
            return push(seg, sn, write)
        return lax.cond(cnt > 0, do, lambda s: s, sn)

    def run_groups(idbuf, ngroups, sn, on_group, do_flush):
        def group_body(g, carry):
            cur, cnt, sn = carry
            i0 = g * 16
            idvec = idbuf[pl.ds(i0, 16)]
            a = idvec[0]
            z = idvec[15]

            def fast(cur, cnt, sn):
                adder = on_group(g)

                def new_seg(sn):
                    sn2 = do_flush(cur, cnt, sn)
                    adder(True)
                    return a, jnp.int32(16), sn2

                def same_seg(sn):
                    adder(False)
                    return cur, cnt + 16, sn
                return lax.cond(a != cur, new_seg, same_seg, sn)

            def slow(cur, cnt, sn):
                for k in range(16):
                    idv = idvec[k]
                    new = idv != cur
                    rowadd = on_group(g, k)

                    def donew(sn, cnt=cnt, cur=cur, rowadd=rowadd):
                        sn2 = do_flush(cur, cnt, sn)
                        rowadd(True)
                        return sn2

                    def doold(sn, rowadd=rowadd):
                        rowadd(False)
                        return sn
                    sn = lax.cond(new, donew, doold, sn)
                    cnt = jnp.where(new, 1, cnt + 1)
                    cur = idv
                return cur, cnt, sn

            return lax.cond(a == z, fast, slow, cur, cnt, sn)

        cur, cnt, sn = lax.fori_loop(
            0, ngroups, group_body, (jnp.int32(-1), jnp.int32(0), sn))
        return do_flush(cur, cnt, sn)

    def process(buf, idbuf, sn):
        def on_group(g, k=None):
            q0 = g * (D // 8)

            def vreg(k, j):
                e = k * D + 16 * j
                return buf[q0 + e // 128, pl.ds(e % 128, 16)]

            if k is None:
                sums = []
                for j in range(nvec):
                    rows = [vreg(k2, j) for k2 in range(16)]
                    while len(rows) > 1:
                        rows = ([rows[p] + rows[p + 1]
                                 for p in range(0, len(rows) - 1, 2)]
                                + (rows[-1:] if len(rows) % 2 else []))
                    sums.append(rows[0])
            else:
                sums = [vreg(k, j) for j in range(nvec)]

            def adder(reset):
                for j in range(nvec):
                    if reset:
                        acc_buf[0, pl.ds(16 * j, 16)] = sums[j]
                    else:
                        acc_buf[0, pl.ds(16 * j, 16)] = (
                            acc_buf[0, pl.ds(16 * j, 16)] + sums[j])
            return adder
        return run_groups(idbuf, BR // 16, sn, on_group, flush)

    sn0 = jnp.int32(0)
    if CBR:
        pltpu.sync_copy(cids_hbm.at[pl.ds(tid * CBR, CBR)],
                        idbufA.at[pl.ds(0, CBR)])
        sn0 = run_groups(idbufA, CBR // 16, sn0,
                         lambda g, k=None: (lambda reset: None), cflush)

    def issue(k, buf, idbuf, sem):
        b = tid + 32 * k
        pltpu.async_copy(data_hbm.at[pl.ds(b * BP, BP)], buf, sem)
        pltpu.async_copy(ids_hbm.at[pl.ds(b * BR, BR)], idbuf, sem)

    def wait(buf, idbuf, sem):
        pltpu.make_async_copy(data_hbm.at[pl.ds(0, BP)], buf, sem).wait()
        pltpu.make_async_copy(ids_hbm.at[pl.ds(0, BR)], idbuf, sem).wait()

    issue(0, bufA, idbufA, semA)
    if KMAX > 1:
        issue(1, bufB, idbufB, semB)

    def pair_body(m, sn):
        k0 = 2 * m
        wait(bufA, idbufA, semA)
        sn = process(bufA, idbufA, sn)
        lax.cond(k0 + 2 < KMAX,
                 lambda: issue(k0 + 2, bufA, idbufA, semA) or 0,
                 lambda: 0)
        wait(bufB, idbufB, semB)
        sn = process(bufB, idbufB, sn)
        lax.cond(k0 + 3 < KMAX,
                 lambda: issue(k0 + 3, bufB, idbufB, semB) or 0,
                 lambda: 0)
        return sn

    sn = lax.fori_loop(0, KMAX // 2, pair_body, sn0)
    if KMAX % 2:
        wait(bufA, idbufA, semA)
        sn = process(bufA, idbufA, sn)

    pltpu.sync_copy(stage_v, table.at[stage_iv.at[0]], add=True)

    plsc.subcore_barrier()
    pltpu.sync_copy(table.at[pl.ds(sid * _ZROWS, _ZROWS)],
                    out_hbm.at[cid, pl.ds(sid * _ZROWS, _ZROWS)])


def _sc_pool(data_packed, ids, d, block_packed_rows, cids=None):
    np_, dw = data_packed.shape
    assert dw == 128
    bp = block_packed_rows
    br = bp * 128 // d
    nb = np_ // bp
    assert nb * bp == np_ and nb % 32 == 0 and br % 16 == 0
    kmax = nb // 32
    cbr = 0 if cids is None else cids.shape[0] // 32
    mesh = plsc.VectorSubcoreMesh(core_axis_name="c", subcore_axis_name="s")
    body = functools.partial(
        _sc_pool_body, BP=bp, BR=br, D=d, KMAX=kmax,
        CNT=(d + 32 <= _W), CBR=cbr)
    args = (data_packed, ids) if cids is None else (data_packed, ids, cids)
    return pl.kernel(
        body,
        out_type=jax.ShapeDtypeStruct((2, _GP, _W), jnp.float32),
        mesh=mesh,
        compiler_params=pltpu.CompilerParams(use_tc_tiling_on_sc=False),
        scratch_types=[
            pltpu.VMEM((bp, 128), jnp.float32),
            pltpu.VMEM((bp, 128), jnp.float32),
            pltpu.VMEM((max(br, cbr),), jnp.int32),
            pltpu.VMEM((br,), jnp.int32),
            pltpu.VMEM((_ZCHUNK, _W), jnp.float32),
            pltpu.VMEM((_SCAP, _W), jnp.float32),
            pltpu.VMEM((1, 16), jnp.int32),
            pltpu.VMEM((1, d), jnp.float32),
            pltpu.VMEM_SHARED((_GP, _W), jnp.float32),
            pltpu.SemaphoreType.DMA,
            pltpu.SemaphoreType.DMA,
        ],
    )(*args)


def _mlp_body(pb_ref, ps_ref, st_ref,
              w1b_ref, w1s_ref, w1t_ref, b1_ref,
              w2_ref, b2_ref, w3_ref, b3_ref, out_ref):
    pb = pb_ref[0] + pb_ref[1]
    ps = ps_ref[0] + ps_ref[1]
    bp = pb[:_G, :16] / jnp.maximum(pb[:_G, 16:17], 1.0)
    sp = ps[:_G, :128] / jnp.maximum(pb[:_G, 17:18], 1.0)

    def dot(a, b):
        return jax.lax.dot_general(a, b, (((1,), (0,)), ((), ())),
                                   preferred_element_type=jnp.float32)

    h = dot(bp, w1b_ref[...]) + dot(sp, w1s_ref[...]) + dot(st_ref[...], w1t_ref[...])
    h = jnp.maximum(h + b1_ref[...], 0.0)
    h = jnp.maximum(dot(h, w2_ref[...]) + b2_ref[...], 0.0)
    out_ref[...] = jnp.maximum(dot(h, w3_ref[...]) + b3_ref[...], 0.0)


def kernel(sites, bonds, states, graph_to_sites, graph_to_bonds,
           W1, b1, W2, b2, W3, b3):
    num_graphs, state_len = states.shape
    bond_len = bonds.shape[1]
    site_len = sites.shape[1]
    assert num_graphs == _G and bond_len == 16 and site_len == 128

    n_sites = sites.shape[0]

    bonds_packed = bonds.reshape(-1, 128)
    cpad = (-n_sites) % (32 * 16)
    cids = jnp.pad(graph_to_sites, (0, cpad), constant_values=_G)
    pb = _sc_pool(bonds_packed, graph_to_bonds, 16, 250, cids)

    pad = (-n_sites) % (32 * 320)
    sites_p = jnp.pad(sites, ((0, pad), (0, 0)))
    ids_s = jnp.pad(graph_to_sites, (0, pad), constant_values=_G)
    ps = _sc_pool(sites_p, ids_s, 128, 320)

    w1 = W1.T
    w1b = w1[:bond_len]
    w1s = w1[bond_len:bond_len + site_len]
    w1t = w1[bond_len + site_len:]

    out = pl.pallas_call(
        _mlp_body,
        out_shape=jax.ShapeDtypeStruct((num_graphs, state_len), jnp.float32),
    )(pb, ps, states,
      w1b, w1s, w1t, b1.reshape(1, -1),
      W2.T, b2.reshape(1, -1), W3.T, b3.reshape(1, -1))
    return out
